# baseline (speedup 1.0000x reference)
import functools

import jax
import jax.numpy as jnp
from jax.experimental import pallas as pl
from jax.experimental.pallas import tpu as pltpu

_N = 4096
_TP = 256
_KP = 8


def _chamfer_body(p_ref, g_ref, o1_ref, o2_ref):
    g = g_ref[0]
    yy = jnp.sum(g * g, axis=0, keepdims=True)

    def step(c, carry):
        cacc, s1 = carry
        pc = p_ref[0, pl.ds(c * _TP, _TP), :]
        xxc = jnp.sum(pc * pc, axis=1, keepdims=True)
        zz = jax.lax.dot_general(
            pc, g, (((1,), (0,)), ((), ())),
            preferred_element_type=jnp.float32)
        d = (xxc + yy) - 2.0 * zz
        s1 = s1 + jnp.sum(jnp.min(d, axis=1))
        cacc = jnp.minimum(cacc, jnp.min(d, axis=0, keepdims=True))
        return cacc, s1

    cacc0 = jnp.full((1, _N), jnp.inf, dtype=jnp.float32)
    cacc, s1 = jax.lax.fori_loop(0, _N // _TP, step, (cacc0, jnp.float32(0.0)))
    s2 = jnp.sum(cacc)
    o1_ref[0, 0, :] = jnp.full((128,), s1, dtype=jnp.float32)
    o2_ref[0, 0, :] = jnp.full((128,), s2, dtype=jnp.float32)


def _voxelize(coord):
    coord_no_nan = jnp.where(jnp.isnan(coord), jnp.inf, coord)
    global_min = jnp.min(coord_no_nan, axis=1, keepdims=True)
    grid_coord = (coord - global_min) / 0.1
    return grid_coord.astype(jnp.int32).astype(jnp.float32)


@jax.jit
def kernel(preds, gts):
    pv = _voxelize(preds)
    gv = _voxelize(gts)
    p8 = jnp.concatenate([preds, pv], axis=0)
    g8 = jnp.concatenate([gts, gv], axis=0)
    p8 = jnp.pad(p8, ((0, 0), (0, 0), (0, _KP - 3)))
    g8 = jnp.pad(g8, ((0, 0), (0, 0), (0, _KP - 3))).transpose(0, 2, 1)

    s1, s2 = pl.pallas_call(
        _chamfer_body,
        grid=(8,),
        in_specs=[
            pl.BlockSpec((1, _N, _KP), lambda b: (b, 0, 0)),
            pl.BlockSpec((1, _KP, _N), lambda b: (b, 0, 0)),
        ],
        out_specs=[
            pl.BlockSpec((1, 1, 128), lambda b: (b, 0, 0)),
            pl.BlockSpec((1, 1, 128), lambda b: (b, 0, 0)),
        ],
        out_shape=[
            jax.ShapeDtypeStruct((8, 1, 128), jnp.float32),
            jax.ShapeDtypeStruct((8, 1, 128), jnp.float32),
        ],
    )(p8, g8)

    total = jnp.sum(s1[:, 0, 0]) + jnp.sum(s2[:, 0, 0])
    return total / jnp.float32(4 * _N)


# speedup vs baseline: 1.2432x; 1.2432x over previous
import functools

import jax
import jax.numpy as jnp
from jax.experimental import pallas as pl
from jax.experimental.pallas import tpu as pltpu

_N = 4096
_TP = 256
_KA = 8


def _chamfer_body(p_ref, g_ref, o1_ref, o2_ref):
    ga = g_ref[0]

    def step(c, carry):
        cacc, s1 = carry
        pc = p_ref[0, pl.ds(c * _TP, _TP), :]
        d = jax.lax.dot_general(
            pc, ga, (((1,), (0,)), ((), ())),
            preferred_element_type=jnp.float32)
        s1 = s1 + jnp.sum(jnp.min(d, axis=1))
        cacc = jnp.minimum(cacc, jnp.min(d, axis=0, keepdims=True))
        return cacc, s1

    cacc0 = jnp.full((1, _N), jnp.inf, dtype=jnp.float32)
    cacc, s1 = jax.lax.fori_loop(0, _N // _TP, step, (cacc0, jnp.float32(0.0)))
    s2 = jnp.sum(cacc)
    o1_ref[0, 0, :] = jnp.full((128,), s1, dtype=jnp.float32)
    o2_ref[0, 0, :] = jnp.full((128,), s2, dtype=jnp.float32)


def _voxelize(coord):
    coord_no_nan = jnp.where(jnp.isnan(coord), jnp.inf, coord)
    global_min = jnp.min(coord_no_nan, axis=1, keepdims=True)
    grid_coord = (coord - global_min) / 0.1
    return grid_coord.astype(jnp.int32).astype(jnp.float32)


@jax.jit
def kernel(preds, gts):
    pv = _voxelize(preds)
    gv = _voxelize(gts)
    p8 = jnp.concatenate([preds, pv], axis=0)
    g8 = jnp.concatenate([gts, gv], axis=0)

    xx = jnp.sum(p8 * p8, axis=2, keepdims=True)
    yy = jnp.sum(g8 * g8, axis=2, keepdims=True)
    ones = jnp.ones((8, _N, 1), jnp.float32)
    zeros = jnp.zeros((8, _N, _KA - 5), jnp.float32)
    pa8 = jnp.concatenate([p8, xx, ones, zeros], axis=2)
    ga8 = jnp.concatenate([-2.0 * g8, ones, yy, zeros], axis=2)
    ga8 = ga8.transpose(0, 2, 1)

    s1, s2 = pl.pallas_call(
        _chamfer_body,
        grid=(8,),
        in_specs=[
            pl.BlockSpec((1, _N, _KA), lambda b: (b, 0, 0)),
            pl.BlockSpec((1, _KA, _N), lambda b: (b, 0, 0)),
        ],
        out_specs=[
            pl.BlockSpec((1, 1, 128), lambda b: (b, 0, 0)),
            pl.BlockSpec((1, 1, 128), lambda b: (b, 0, 0)),
        ],
        out_shape=[
            jax.ShapeDtypeStruct((8, 1, 128), jnp.float32),
            jax.ShapeDtypeStruct((8, 1, 128), jnp.float32),
        ],
    )(pa8, ga8)

    total = jnp.sum(s1[:, 0, 0]) + jnp.sum(s2[:, 0, 0])
    return total / jnp.float32(4 * _N)
